# Initial kernel scaffold; baseline (speedup 1.0000x reference)
import functools

import jax
import jax.numpy as jnp
from jax import lax
from jax.experimental import pallas as pl
from jax.experimental.pallas import tpu as pltpu
from jax.experimental.pallas import tpu_sc as plsc

N = 10000
E = 320000
NC, NS = 2, 16
NW = NC * NS
EW = E // NW
B = 125
NB = EW // B
RPS = N // NS
BN = 1000

_MESH = plsc.VectorSubcoreMesh(core_axis_name="c", subcore_axis_name="s")


@functools.partial(
    pl.kernel,
    out_type=jax.ShapeDtypeStruct((NW, N), jnp.float32),
    mesh=_MESH,
    scratch_types=[pltpu.VMEM((EW,), jnp.int32), pltpu.VMEM((N,), jnp.float32)],
)
def _deg_hist(dst_hbm, hist_hbm, idx_v, hist_v):
    w = lax.axis_index("s") * NC + lax.axis_index("c")
    pltpu.sync_copy(dst_hbm.at[w], idx_v)
    zeros = jnp.zeros((16,), jnp.float32)
    ones = jnp.ones((16,), jnp.float32)

    def zero_body(i, carry):
        hist_v[pl.ds(i * 16, 16)] = zeros
        return carry

    lax.fori_loop(0, N // 16, zero_body, 0)

    def acc_body(i, carry):
        d = idx_v[pl.ds(i * 16, 16)]
        plsc.addupdate_scatter(hist_v, [d], ones)
        return carry

    lax.fori_loop(0, EW // 16, acc_body, 0)
    pltpu.sync_copy(hist_v, hist_hbm.at[w])


def _make_agg(C, Fc):
    def body(*refs):
        ztabs = refs[:C]
        src3, dst3, zrow = refs[C:C + 3]
        p_out = refs[C + 3]
        ivs, ivd, rows, sem, acc = refs[C + 4:]
        cc = lax.axis_index("c")
        ss = lax.axis_index("s")
        w = ss * NC + cc
        pltpu.sync_copy(src3.at[w], ivs)
        pltpu.sync_copy(dst3.at[w], ivd)
        for c in range(C):
            pltpu.sync_copy(zrow, acc.at[pl.ds(ss * RPS, RPS)])
            plsc.subcore_barrier()

            def batch(j, carry, _zt=ztabs[c]):
                pltpu.async_copy(_zt.at[ivs.at[j]], rows, sem).wait()
                pltpu.sync_copy(rows, acc.at[ivd.at[j]], add=True)
                return carry

            lax.fori_loop(0, NB, batch, 0)
            plsc.subcore_barrier()
            pltpu.sync_copy(acc.at[pl.ds(ss * RPS, RPS)],
                            p_out.at[cc, c, pl.ds(ss * RPS, RPS)])

    return pl.kernel(
        body,
        out_type=jax.ShapeDtypeStruct((NC, C, N, Fc), jnp.float32),
        mesh=_MESH,
        scratch_types=[
            pltpu.VMEM((NB, B), jnp.int32),
            pltpu.VMEM((NB, B), jnp.int32),
            pltpu.VMEM((B, Fc), jnp.float32),
            pltpu.SemaphoreType.DMA,
            pltpu.VMEM_SHARED((N, Fc), jnp.float32),
        ],
    )


_agg_l1 = _make_agg(1, 128)
_agg_l2 = _make_agg(4, 128)
_agg_l3 = _make_agg(2, 128)
_agg_l4 = _make_agg(1, 64)


def _tc_prep(hist, x):
    def body(h_ref, x_ref, dinv_ref, z_ref):
        deg = jnp.sum(h_ref[...], axis=0) + 1.0
        dv = lax.rsqrt(deg)
        dinv_ref[...] = dv[:, None]
        z_ref[...] = x_ref[...] * dv[:, None]

    return pl.pallas_call(
        body,
        grid=(N // BN,),
        in_specs=[pl.BlockSpec((NW, BN), lambda i: (0, i)),
                  pl.BlockSpec((BN, 128), lambda i: (i, 0))],
        out_specs=[pl.BlockSpec((BN, 1), lambda i: (i, 0)),
                   pl.BlockSpec((BN, 128), lambda i: (i, 0))],
        out_shape=[jax.ShapeDtypeStruct((N, 1), jnp.float32),
                   jax.ShapeDtypeStruct((N, 128), jnp.float32)],
    )(hist, x)


def _tc_l1(p1, z1, dinv, w1, b1):
    def body(p_ref, z_ref, d_ref, w_ref, b_ref, o_ref):
        s = p_ref[0, 0] + p_ref[1, 0] + z_ref[...]
        y = s * d_ref[...]
        h = jnp.dot(y, w_ref[...], preferred_element_type=jnp.float32)
        h = jnp.maximum(h + b_ref[...], 0.0)
        o_ref[0] = h * d_ref[...]

    return pl.pallas_call(
        body,
        grid=(N // BN, 4),
        in_specs=[pl.BlockSpec((2, 1, BN, 128), lambda i, j: (0, 0, i, 0)),
                  pl.BlockSpec((BN, 128), lambda i, j: (i, 0)),
                  pl.BlockSpec((BN, 1), lambda i, j: (i, 0)),
                  pl.BlockSpec((128, 128), lambda i, j: (0, j)),
                  pl.BlockSpec((128,), lambda i, j: (j,))],
        out_specs=pl.BlockSpec((1, BN, 128), lambda i, j: (j, i, 0)),
        out_shape=jax.ShapeDtypeStruct((4, N, 128), jnp.float32),
    )(p1, z1, dinv, w1, b1)


def _tc_l2(p2, z2tab, dinv, w2, b2, w3):
    def body(p_ref, z_ref, d_ref, w2_ref, b2_ref, w3_ref, o_ref):
        dv = d_ref[...]
        w2m = w2_ref[...]
        acc = jnp.zeros((BN, 512), jnp.float32)
        for c in range(4):
            s = p_ref[0, c] + p_ref[1, c] + z_ref[c]
            acc = acc + jnp.dot(s * dv, w2m[c * 128:(c + 1) * 128, :],
                                preferred_element_type=jnp.float32)
        h2 = jnp.maximum(acc + b2_ref[...], 0.0)
        g3 = jnp.dot(h2, w3_ref[...], preferred_element_type=jnp.float32)
        z3 = g3 * dv
        o_ref[0] = z3[:, :128]
        o_ref[1] = z3[:, 128:]

    return pl.pallas_call(
        body,
        grid=(N // BN,),
        in_specs=[pl.BlockSpec((2, 4, BN, 128), lambda i: (0, 0, i, 0)),
                  pl.BlockSpec((4, BN, 128), lambda i: (0, i, 0)),
                  pl.BlockSpec((BN, 1), lambda i: (i, 0)),
                  pl.BlockSpec((512, 512), lambda i: (0, 0)),
                  pl.BlockSpec((512,), lambda i: (0,)),
                  pl.BlockSpec((512, 256), lambda i: (0, 0))],
        out_specs=pl.BlockSpec((2, BN, 128), lambda i: (0, i, 0)),
        out_shape=jax.ShapeDtypeStruct((2, N, 128), jnp.float32),
    )(p2, z2tab, dinv, w2, b2, w3)


def _tc_l3(p3, z3tab, dinv, b3, w4):
    def body(p_ref, z_ref, d_ref, b3_ref, w4_ref, o_ref):
        dv = d_ref[...]
        b3v = b3_ref[...]
        w4m = w4_ref[...]
        g4 = jnp.zeros((BN, 64), jnp.float32)
        for c in range(2):
            s = p_ref[0, c] + p_ref[1, c] + z_ref[c]
            h3c = jnp.maximum(s * dv + b3v[c * 128:(c + 1) * 128], 0.0)
            g4 = g4 + jnp.dot(h3c, w4m[c * 128:(c + 1) * 128, :],
                              preferred_element_type=jnp.float32)
        o_ref[...] = g4 * dv

    return pl.pallas_call(
        body,
        grid=(N // BN,),
        in_specs=[pl.BlockSpec((2, 2, BN, 128), lambda i: (0, 0, i, 0)),
                  pl.BlockSpec((2, BN, 128), lambda i: (0, i, 0)),
                  pl.BlockSpec((BN, 1), lambda i: (i, 0)),
                  pl.BlockSpec((256,), lambda i: (0,)),
                  pl.BlockSpec((256, 64), lambda i: (0, 0))],
        out_specs=pl.BlockSpec((BN, 64), lambda i: (i, 0)),
        out_shape=jax.ShapeDtypeStruct((N, 64), jnp.float32),
    )(p3, z3tab, dinv, b3, w4)


def _tc_l4(p4, z4, dinv, b4):
    def body(p_ref, z_ref, d_ref, b4_ref, o_ref):
        s = p_ref[0, 0] + p_ref[1, 0] + z_ref[...]
        o_ref[...] = jnp.maximum(s * d_ref[...] + b4_ref[...], 0.0)

    return pl.pallas_call(
        body,
        grid=(N // BN,),
        in_specs=[pl.BlockSpec((2, 1, BN, 64), lambda i: (0, 0, i, 0)),
                  pl.BlockSpec((BN, 64), lambda i: (i, 0)),
                  pl.BlockSpec((BN, 1), lambda i: (i, 0)),
                  pl.BlockSpec((64,), lambda i: (0,))],
        out_specs=pl.BlockSpec((BN, 64), lambda i: (i, 0)),
        out_shape=jax.ShapeDtypeStruct((N, 64), jnp.float32),
    )(p4, z4, dinv, b4)


def kernel(x, edge_index, W1, b1, W2, b2, W3, b3, W4, b4):
    ei = edge_index.astype(jnp.int32)
    src3 = ei[0].reshape(NW, NB, B)
    dst3 = ei[1].reshape(NW, NB, B)
    dst2 = ei[1].reshape(NW, EW)
    zrow128 = jnp.zeros((RPS, 128), jnp.float32)
    zrow64 = jnp.zeros((RPS, 64), jnp.float32)

    hist = _deg_hist(dst2)
    dinv, z1 = _tc_prep(hist, x)

    p1 = _agg_l1(z1, src3, dst3, zrow128)
    z2tab = _tc_l1(p1, z1, dinv, W1, b1)

    p2 = _agg_l2(z2tab[0], z2tab[1], z2tab[2], z2tab[3], src3, dst3, zrow128)
    z3tab = _tc_l2(p2, z2tab, dinv, W2, b2, W3)

    p3 = _agg_l3(z3tab[0], z3tab[1], src3, dst3, zrow128)
    z4 = _tc_l3(p3, z3tab, dinv, b3, W4)

    p4 = _agg_l4(z4, src3, dst3, zrow64)
    return _tc_l4(p4, z4, dinv, b4)


# speedup vs baseline: 14.7193x; 14.7193x over previous
import functools

import jax
import jax.numpy as jnp
from jax import lax
from jax.experimental import pallas as pl
from jax.experimental.pallas import tpu as pltpu
from jax.experimental.pallas import tpu_sc as plsc

N = 10000
E = 320000
NC, NS = 2, 16
NW = NC * NS
EW = E // NW
B = 125
NB = EW // B
RPS = N // NS
BN = 1000

_MESH = plsc.VectorSubcoreMesh(core_axis_name="c", subcore_axis_name="s")


@functools.partial(
    pl.kernel,
    out_type=jax.ShapeDtypeStruct((NW, N), jnp.float32),
    mesh=_MESH,
    scratch_types=[pltpu.VMEM((EW,), jnp.int32), pltpu.VMEM((N,), jnp.float32)],
    compiler_params=pltpu.CompilerParams(needs_layout_passes=False),
)
def _deg_hist(dst_hbm, hist_hbm, idx_v, hist_v):
    w = lax.axis_index("s") * NC + lax.axis_index("c")
    pltpu.sync_copy(dst_hbm.at[w], idx_v)
    zeros = jnp.zeros((16,), jnp.float32)
    ones = jnp.ones((16,), jnp.float32)

    def zero_body(i, carry):
        hist_v[pl.ds(i * 16, 16)] = zeros
        return carry

    lax.fori_loop(0, N // 16, zero_body, 0)

    def acc_body(i, carry):
        d = idx_v[pl.ds(i * 16, 16)]
        plsc.addupdate_scatter(hist_v, [d], ones)
        return carry

    lax.fori_loop(0, EW // 16, acc_body, 0)
    pltpu.sync_copy(hist_v, hist_hbm.at[w])


def _make_agg(C, Fc):
    def body(*refs):
        ztabs = refs[:C]
        src3, dst3, zrow = refs[C:C + 3]
        p_out = refs[C + 3]
        ivs, ivd, rows, sem, acc = refs[C + 4:]
        cc = lax.axis_index("c")
        ss = lax.axis_index("s")
        w = ss * NC + cc
        pltpu.sync_copy(src3.at[w], ivs)
        pltpu.sync_copy(dst3.at[w], ivd)
        for c in range(C):
            pltpu.sync_copy(zrow, acc.at[pl.ds(ss * RPS, RPS)])
            plsc.subcore_barrier()

            def batch(j, carry, _zt=ztabs[c]):
                pltpu.async_copy(_zt.at[ivs.at[j]], rows, sem).wait()
                pltpu.sync_copy(rows, acc.at[ivd.at[j]], add=True)
                return carry

            lax.fori_loop(0, NB, batch, 0)
            plsc.subcore_barrier()
            pltpu.sync_copy(acc.at[pl.ds(ss * RPS, RPS)],
                            p_out.at[cc, c, pl.ds(ss * RPS, RPS)])

    return pl.kernel(
        body,
        out_type=jax.ShapeDtypeStruct((NC, C, N, Fc), jnp.float32),
        mesh=_MESH,
        compiler_params=pltpu.CompilerParams(use_tc_tiling_on_sc=False),
        scratch_types=[
            pltpu.VMEM((NB, B), jnp.int32),
            pltpu.VMEM((NB, B), jnp.int32),
            pltpu.VMEM((B, Fc), jnp.float32),
            pltpu.SemaphoreType.DMA,
            pltpu.VMEM_SHARED((N, Fc), jnp.float32),
        ],
    )


_agg_l1 = _make_agg(1, 128)
_agg_l2 = _make_agg(4, 128)
_agg_l3 = _make_agg(2, 128)
_agg_l4 = _make_agg(1, 64)


def _tc_prep(hist_t, x):
    def body(h_ref, x_ref, dinv_ref, z_ref):
        deg = jnp.sum(h_ref[...], axis=1, keepdims=True) + 1.0
        dv = lax.rsqrt(deg)
        dinv_ref[...] = dv
        z_ref[...] = x_ref[...] * dv

    return pl.pallas_call(
        body,
        grid=(N // BN,),
        in_specs=[pl.BlockSpec((BN, NW), lambda i: (i, 0)),
                  pl.BlockSpec((BN, 128), lambda i: (i, 0))],
        out_specs=[pl.BlockSpec((BN, 1), lambda i: (i, 0)),
                   pl.BlockSpec((BN, 128), lambda i: (i, 0))],
        out_shape=[jax.ShapeDtypeStruct((N, 1), jnp.float32),
                   jax.ShapeDtypeStruct((N, 128), jnp.float32)],
    )(hist_t, x)


def _tc_l1(p1, z1, dinv, w1, b1):
    def body(p_ref, z_ref, d_ref, w_ref, b_ref, o_ref):
        s = p_ref[0, 0] + p_ref[1, 0] + z_ref[...]
        y = s * d_ref[...]
        h = jnp.dot(y, w_ref[...], preferred_element_type=jnp.float32)
        h = jnp.maximum(h + b_ref[...], 0.0)
        o_ref[0] = h * d_ref[...]

    return pl.pallas_call(
        body,
        grid=(N // BN, 4),
        in_specs=[pl.BlockSpec((2, 1, BN, 128), lambda i, j: (0, 0, i, 0)),
                  pl.BlockSpec((BN, 128), lambda i, j: (i, 0)),
                  pl.BlockSpec((BN, 1), lambda i, j: (i, 0)),
                  pl.BlockSpec((128, 128), lambda i, j: (0, j)),
                  pl.BlockSpec((128,), lambda i, j: (j,))],
        out_specs=pl.BlockSpec((1, BN, 128), lambda i, j: (j, i, 0)),
        out_shape=jax.ShapeDtypeStruct((4, N, 128), jnp.float32),
    )(p1, z1, dinv, w1, b1)


def _tc_l2(p2, z2tab, dinv, w2, b2, w3):
    def body(p_ref, z_ref, d_ref, w2_ref, b2_ref, w3_ref, o_ref):
        dv = d_ref[...]
        w2m = w2_ref[...]
        acc = jnp.zeros((BN, 512), jnp.float32)
        for c in range(4):
            s = p_ref[0, c] + p_ref[1, c] + z_ref[c]
            acc = acc + jnp.dot(s * dv, w2m[c * 128:(c + 1) * 128, :],
                                preferred_element_type=jnp.float32)
        h2 = jnp.maximum(acc + b2_ref[...], 0.0)
        g3 = jnp.dot(h2, w3_ref[...], preferred_element_type=jnp.float32)
        z3 = g3 * dv
        o_ref[0] = z3[:, :128]
        o_ref[1] = z3[:, 128:]

    return pl.pallas_call(
        body,
        grid=(N // BN,),
        in_specs=[pl.BlockSpec((2, 4, BN, 128), lambda i: (0, 0, i, 0)),
                  pl.BlockSpec((4, BN, 128), lambda i: (0, i, 0)),
                  pl.BlockSpec((BN, 1), lambda i: (i, 0)),
                  pl.BlockSpec((512, 512), lambda i: (0, 0)),
                  pl.BlockSpec((512,), lambda i: (0,)),
                  pl.BlockSpec((512, 256), lambda i: (0, 0))],
        out_specs=pl.BlockSpec((2, BN, 128), lambda i: (0, i, 0)),
        out_shape=jax.ShapeDtypeStruct((2, N, 128), jnp.float32),
    )(p2, z2tab, dinv, w2, b2, w3)


def _tc_l3(p3, z3tab, dinv, b3, w4):
    def body(p_ref, z_ref, d_ref, b3_ref, w4_ref, o_ref):
        dv = d_ref[...]
        b3v = b3_ref[...]
        w4m = w4_ref[...]
        g4 = jnp.zeros((BN, 64), jnp.float32)
        for c in range(2):
            s = p_ref[0, c] + p_ref[1, c] + z_ref[c]
            h3c = jnp.maximum(s * dv + b3v[c * 128:(c + 1) * 128], 0.0)
            g4 = g4 + jnp.dot(h3c, w4m[c * 128:(c + 1) * 128, :],
                              preferred_element_type=jnp.float32)
        o_ref[...] = g4 * dv

    return pl.pallas_call(
        body,
        grid=(N // BN,),
        in_specs=[pl.BlockSpec((2, 2, BN, 128), lambda i: (0, 0, i, 0)),
                  pl.BlockSpec((2, BN, 128), lambda i: (0, i, 0)),
                  pl.BlockSpec((BN, 1), lambda i: (i, 0)),
                  pl.BlockSpec((256,), lambda i: (0,)),
                  pl.BlockSpec((256, 64), lambda i: (0, 0))],
        out_specs=pl.BlockSpec((BN, 64), lambda i: (i, 0)),
        out_shape=jax.ShapeDtypeStruct((N, 64), jnp.float32),
    )(p3, z3tab, dinv, b3, w4)


def _tc_l4(p4, z4, dinv, b4):
    def body(p_ref, z_ref, d_ref, b4_ref, o_ref):
        s = p_ref[0, 0] + p_ref[1, 0] + z_ref[...]
        o_ref[...] = jnp.maximum(s * d_ref[...] + b4_ref[...], 0.0)

    return pl.pallas_call(
        body,
        grid=(N // BN,),
        in_specs=[pl.BlockSpec((2, 1, BN, 64), lambda i: (0, 0, i, 0)),
                  pl.BlockSpec((BN, 64), lambda i: (i, 0)),
                  pl.BlockSpec((BN, 1), lambda i: (i, 0)),
                  pl.BlockSpec((64,), lambda i: (0,))],
        out_specs=pl.BlockSpec((BN, 64), lambda i: (i, 0)),
        out_shape=jax.ShapeDtypeStruct((N, 64), jnp.float32),
    )(p4, z4, dinv, b4)


def kernel(x, edge_index, W1, b1, W2, b2, W3, b3, W4, b4):
    ei = edge_index.astype(jnp.int32)
    src3 = ei[0].reshape(NW, NB, B)
    dst3 = ei[1].reshape(NW, NB, B)
    dst2 = ei[1].reshape(NW, EW)
    zrow128 = jnp.zeros((RPS, 128), jnp.float32)
    zrow64 = jnp.zeros((RPS, 64), jnp.float32)

    hist = _deg_hist(dst2)
    dinv, z1 = _tc_prep(hist.T, x)

    p1 = _agg_l1(z1, src3, dst3, zrow128)
    z2tab = _tc_l1(p1, z1, dinv, W1, b1)

    p2 = _agg_l2(z2tab[0], z2tab[1], z2tab[2], z2tab[3], src3, dst3, zrow128)
    z3tab = _tc_l2(p2, z2tab, dinv, W2, b2, W3)

    p3 = _agg_l3(z3tab[0], z3tab[1], src3, dst3, zrow128)
    z4 = _tc_l3(p3, z3tab, dinv, b3, W4)

    p4 = _agg_l4(z4, src3, dst3, zrow64)
    return _tc_l4(p4, z4, dinv, b4)


# speedup vs baseline: 21.7214x; 1.4757x over previous
import functools

import jax
import jax.numpy as jnp
from jax import lax
from jax.experimental import pallas as pl
from jax.experimental.pallas import tpu as pltpu
from jax.experimental.pallas import tpu_sc as plsc

N = 10000
E = 320000
NC, NS = 2, 16
NW = NC * NS
EW = E // NW
B = 100
NB = EW // B
RPS = N // NS
BN = 1000

_MESH = plsc.VectorSubcoreMesh(core_axis_name="c", subcore_axis_name="s")


@functools.partial(
    pl.kernel,
    out_type=jax.ShapeDtypeStruct((NW, N), jnp.float32),
    mesh=_MESH,
    scratch_types=[pltpu.VMEM((EW,), jnp.int32), pltpu.VMEM((N,), jnp.float32)],
    compiler_params=pltpu.CompilerParams(needs_layout_passes=False),
)
def _deg_hist(dst_hbm, hist_hbm, idx_v, hist_v):
    w = lax.axis_index("s") * NC + lax.axis_index("c")
    pltpu.sync_copy(dst_hbm.at[w], idx_v)
    zeros = jnp.zeros((16,), jnp.float32)
    ones = jnp.ones((16,), jnp.float32)

    def zero_body(i, carry):
        hist_v[pl.ds(i * 16, 16)] = zeros
        return carry

    lax.fori_loop(0, N // 16, zero_body, 0)

    def acc_body(i, carry):
        d = idx_v[pl.ds(i * 16, 16)]
        plsc.addupdate_scatter(hist_v, [d], ones)
        return carry

    lax.fori_loop(0, EW // 16, acc_body, 0)
    pltpu.sync_copy(hist_v, hist_hbm.at[w])


def _make_agg(C, Fc):
    def body(*refs):
        ztabs = refs[:C]
        src3, dst3, zrow = refs[C:C + 3]
        p_out = refs[C + 3]
        ivs, ivd, rows0, rows1, sem0, sem1, acc = refs[C + 4:]
        rows = (rows0, rows1)
        sems = (sem0, sem1)
        cc = lax.axis_index("c")
        ss = lax.axis_index("s")
        w = ss * NC + cc
        pltpu.sync_copy(src3.at[w], ivs)
        pltpu.sync_copy(dst3.at[w], ivd)
        for c in range(C):
            zt = ztabs[c]
            pltpu.sync_copy(zrow, acc.at[pl.ds(ss * RPS, RPS)])
            plsc.subcore_barrier()

            pltpu.async_copy(zt.at[ivs.at[0]], rows0, sem0)

            def outer(g, carry):
                for b in range(2):
                    jj = g * 2 + b

                    @pl.when(jj + 1 < NB)
                    def _prefetch():
                        pltpu.async_copy(zt.at[ivs.at[jj + 1]],
                                         rows[1 - b], sems[1 - b])

                    pltpu.make_async_copy(zt.at[ivs.at[jj]],
                                          rows[b], sems[b]).wait()
                    pltpu.sync_copy(rows[b], acc.at[ivd.at[jj]], add=True)
                return carry

            lax.fori_loop(0, NB // 2, outer, 0)
            plsc.subcore_barrier()
            pltpu.sync_copy(acc.at[pl.ds(ss * RPS, RPS)],
                            p_out.at[cc, c, pl.ds(ss * RPS, RPS)])

    return pl.kernel(
        body,
        out_type=jax.ShapeDtypeStruct((NC, C, N, Fc), jnp.float32),
        mesh=_MESH,
        compiler_params=pltpu.CompilerParams(use_tc_tiling_on_sc=False),
        scratch_types=[
            pltpu.VMEM((NB, B), jnp.int32),
            pltpu.VMEM((NB, B), jnp.int32),
            pltpu.VMEM((B, Fc), jnp.float32),
            pltpu.VMEM((B, Fc), jnp.float32),
            pltpu.SemaphoreType.DMA,
            pltpu.SemaphoreType.DMA,
            pltpu.VMEM_SHARED((N, Fc), jnp.float32),
        ],
    )


_agg_l1 = _make_agg(1, 128)
_agg_l2 = _make_agg(4, 128)
_agg_l3 = _make_agg(2, 128)
_agg_l4 = _make_agg(1, 64)


def _tc_prep(hist_t, x):
    def body(h_ref, x_ref, dinv_ref, z_ref):
        deg = jnp.sum(h_ref[...], axis=1, keepdims=True) + 1.0
        dv = lax.rsqrt(deg)
        dinv_ref[...] = dv
        z_ref[...] = x_ref[...] * dv

    return pl.pallas_call(
        body,
        grid=(N // BN,),
        in_specs=[pl.BlockSpec((BN, NW), lambda i: (i, 0)),
                  pl.BlockSpec((BN, 128), lambda i: (i, 0))],
        out_specs=[pl.BlockSpec((BN, 1), lambda i: (i, 0)),
                   pl.BlockSpec((BN, 128), lambda i: (i, 0))],
        out_shape=[jax.ShapeDtypeStruct((N, 1), jnp.float32),
                   jax.ShapeDtypeStruct((N, 128), jnp.float32)],
    )(hist_t, x)


def _tc_l1(p1, z1, dinv, w1, b1):
    def body(p_ref, z_ref, d_ref, w_ref, b_ref, o_ref):
        s = p_ref[0, 0] + p_ref[1, 0] + z_ref[...]
        y = s * d_ref[...]
        h = jnp.dot(y, w_ref[...], preferred_element_type=jnp.float32)
        h = jnp.maximum(h + b_ref[...], 0.0)
        o_ref[0] = h * d_ref[...]

    return pl.pallas_call(
        body,
        grid=(N // BN, 4),
        in_specs=[pl.BlockSpec((2, 1, BN, 128), lambda i, j: (0, 0, i, 0)),
                  pl.BlockSpec((BN, 128), lambda i, j: (i, 0)),
                  pl.BlockSpec((BN, 1), lambda i, j: (i, 0)),
                  pl.BlockSpec((128, 128), lambda i, j: (0, j)),
                  pl.BlockSpec((128,), lambda i, j: (j,))],
        out_specs=pl.BlockSpec((1, BN, 128), lambda i, j: (j, i, 0)),
        out_shape=jax.ShapeDtypeStruct((4, N, 128), jnp.float32),
    )(p1, z1, dinv, w1, b1)


def _tc_l2(p2, z2tab, dinv, w2, b2, w3):
    def body(p_ref, z_ref, d_ref, w2_ref, b2_ref, w3_ref, o_ref):
        dv = d_ref[...]
        w2m = w2_ref[...]
        acc = jnp.zeros((BN, 512), jnp.float32)
        for c in range(4):
            s = p_ref[0, c] + p_ref[1, c] + z_ref[c]
            acc = acc + jnp.dot(s * dv, w2m[c * 128:(c + 1) * 128, :],
                                preferred_element_type=jnp.float32)
        h2 = jnp.maximum(acc + b2_ref[...], 0.0)
        g3 = jnp.dot(h2, w3_ref[...], preferred_element_type=jnp.float32)
        z3 = g3 * dv
        o_ref[0] = z3[:, :128]
        o_ref[1] = z3[:, 128:]

    return pl.pallas_call(
        body,
        grid=(N // BN,),
        in_specs=[pl.BlockSpec((2, 4, BN, 128), lambda i: (0, 0, i, 0)),
                  pl.BlockSpec((4, BN, 128), lambda i: (0, i, 0)),
                  pl.BlockSpec((BN, 1), lambda i: (i, 0)),
                  pl.BlockSpec((512, 512), lambda i: (0, 0)),
                  pl.BlockSpec((512,), lambda i: (0,)),
                  pl.BlockSpec((512, 256), lambda i: (0, 0))],
        out_specs=pl.BlockSpec((2, BN, 128), lambda i: (0, i, 0)),
        out_shape=jax.ShapeDtypeStruct((2, N, 128), jnp.float32),
    )(p2, z2tab, dinv, w2, b2, w3)


def _tc_l3(p3, z3tab, dinv, b3, w4):
    def body(p_ref, z_ref, d_ref, b3_ref, w4_ref, o_ref):
        dv = d_ref[...]
        b3v = b3_ref[...]
        w4m = w4_ref[...]
        g4 = jnp.zeros((BN, 64), jnp.float32)
        for c in range(2):
            s = p_ref[0, c] + p_ref[1, c] + z_ref[c]
            h3c = jnp.maximum(s * dv + b3v[c * 128:(c + 1) * 128], 0.0)
            g4 = g4 + jnp.dot(h3c, w4m[c * 128:(c + 1) * 128, :],
                              preferred_element_type=jnp.float32)
        o_ref[...] = g4 * dv

    return pl.pallas_call(
        body,
        grid=(N // BN,),
        in_specs=[pl.BlockSpec((2, 2, BN, 128), lambda i: (0, 0, i, 0)),
                  pl.BlockSpec((2, BN, 128), lambda i: (0, i, 0)),
                  pl.BlockSpec((BN, 1), lambda i: (i, 0)),
                  pl.BlockSpec((256,), lambda i: (0,)),
                  pl.BlockSpec((256, 64), lambda i: (0, 0))],
        out_specs=pl.BlockSpec((BN, 64), lambda i: (i, 0)),
        out_shape=jax.ShapeDtypeStruct((N, 64), jnp.float32),
    )(p3, z3tab, dinv, b3, w4)


def _tc_l4(p4, z4, dinv, b4):
    def body(p_ref, z_ref, d_ref, b4_ref, o_ref):
        s = p_ref[0, 0] + p_ref[1, 0] + z_ref[...]
        o_ref[...] = jnp.maximum(s * d_ref[...] + b4_ref[...], 0.0)

    return pl.pallas_call(
        body,
        grid=(N // BN,),
        in_specs=[pl.BlockSpec((2, 1, BN, 64), lambda i: (0, 0, i, 0)),
                  pl.BlockSpec((BN, 64), lambda i: (i, 0)),
                  pl.BlockSpec((BN, 1), lambda i: (i, 0)),
                  pl.BlockSpec((64,), lambda i: (0,))],
        out_specs=pl.BlockSpec((BN, 64), lambda i: (i, 0)),
        out_shape=jax.ShapeDtypeStruct((N, 64), jnp.float32),
    )(p4, z4, dinv, b4)


def kernel(x, edge_index, W1, b1, W2, b2, W3, b3, W4, b4):
    ei = edge_index.astype(jnp.int32)
    src3 = ei[0].reshape(NW, NB, B)
    dst3 = ei[1].reshape(NW, NB, B)
    dst2 = ei[1].reshape(NW, EW)
    zrow128 = jnp.zeros((RPS, 128), jnp.float32)
    zrow64 = jnp.zeros((RPS, 64), jnp.float32)

    hist = _deg_hist(dst2)
    dinv, z1 = _tc_prep(hist.T, x)

    p1 = _agg_l1(z1, src3, dst3, zrow128)
    z2tab = _tc_l1(p1, z1, dinv, W1, b1)

    p2 = _agg_l2(z2tab[0], z2tab[1], z2tab[2], z2tab[3], src3, dst3, zrow128)
    z3tab = _tc_l2(p2, z2tab, dinv, W2, b2, W3)

    p3 = _agg_l3(z3tab[0], z3tab[1], src3, dst3, zrow128)
    z4 = _tc_l3(p3, z3tab, dinv, b3, W4)

    p4 = _agg_l4(z4, src3, dst3, zrow64)
    return _tc_l4(p4, z4, dinv, b4)
